# baseline (speedup 1.0000x reference)
import functools

import numpy as np
import jax
import jax.numpy as jnp
from jax import lax
from jax.experimental import pallas as pl
from jax.experimental.pallas import tpu as pltpu
from jax.experimental.pallas import tpu_sc as plsc

B = 8
NQ = 5000
NT = 1000
NM = 1000
NP = 5008
NV = NP // 16
NOUT = 1024
NBINS = 256
NJOBS = 2 * B

_I32_MIN = np.int32(-(2**31))


def _area_key(x1, y1, x2, y2):
    a = (x2 - x1) * (y2 - y1)
    a = jnp.where(a == 0.0, 0.0, a)
    u = lax.bitcast_convert_type(a, jnp.int32)
    s = u >> 31
    m = u ^ (s | _I32_MIN)
    return ~m


def _sc_sort_body(pred_hbm, targ_hbm, out_hbm,
                  bxin, btin, kin, ka, kb, va, vb, hist, hist2):
    wid = lax.axis_index("s")
    lane = lax.iota(jnp.int32, 16)
    ones = jnp.ones(16, jnp.int32)
    zeros16 = jnp.zeros((16,), jnp.int32)
    neg16 = jnp.full((16,), -1, jnp.int32)

    def clr(h):
        def body(j, c):
            h[pl.ds(j * 16, 16)] = zeros16
            return c
        lax.fori_loop(0, NBINS // 16, body, 0)

    def build_keys(src, n_el):
        nvr = (n_el + 15) // 16

        def body(i, c):
            e = lane + i * 16
            valid = e < n_el
            idx = jnp.where(valid, e, 0) * 4
            x1 = plsc.load_gather(src, [idx])
            y1 = plsc.load_gather(src, [idx + 1])
            x2 = plsc.load_gather(src, [idx + 2])
            y2 = plsc.load_gather(src, [idx + 3])
            key = jnp.where(valid, _area_key(x1, y1, x2, y2), -1)
            kin[pl.ds(i * 16, 16)] = key
            d = lax.shift_right_logical(key, 24)
            plsc.addupdate_scatter(hist, [d], ones)
            return c
        lax.fori_loop(0, nvr, body, 0)

        def padbody(i, c):
            kin[pl.ds(i * 16, 16)] = neg16
            return c
        lax.fori_loop(nvr, NV, padbody, 0)

    clr(hist)

    @pl.when(wid < B)
    def _():
        pltpu.sync_copy(pred_hbm.at[wid], bxin)
        build_keys(bxin, NQ)

    @pl.when(wid >= B)
    def _():
        pltpu.sync_copy(targ_hbm.at[wid - B], btin)
        build_keys(btin, NT)

    def scan_a(j, carry):
        tot, c = carry
        h = hist[pl.ds(j * 16, 16)]
        inc = plsc.cumsum(h) + tot
        c = c + jnp.sum(jnp.where(inc < NM, 1, 0))
        return (tot + jnp.sum(h), c)
    _, cut = lax.fori_loop(
        0, NBINS // 16, scan_a, (jnp.int32(0), jnp.int32(0)))

    clr(hist)

    def compact(i, off):
        k = kin[pl.ds(i * 16, 16)]
        d = lax.shift_right_logical(k, 24)
        m = d <= cut
        plsc.store_compressed(kb.at[pl.ds(off, 16)], k, mask=m)
        plsc.store_compressed(vb.at[pl.ds(off, 16)], lane + i * 16, mask=m)
        d0 = k & 255
        plsc.addupdate_scatter(hist, [d0], ones, mask=m)
        return off + jnp.sum(jnp.where(m, 1, 0))
    off = lax.fori_loop(0, NV, compact, jnp.int32(0))

    kb[pl.ds(off, 16)] = neg16
    vb[pl.ds(off, 16)] = lane + NP
    plsc.addupdate_scatter(hist, [jnp.full((16,), 255, jnp.int32)], ones)
    t2 = off // 16 + 1

    bufs = [
        (kb, vb, ka, va, hist, hist2),
        (ka, va, kb, vb, hist2, hist),
        (kb, vb, ka, va, hist, hist2),
        (ka, va, kb, vb, hist2, hist),
    ]
    for p, (ki, vi, ko, vo, hc, hn) in enumerate(bufs):
        shift = 8 * p
        if p < 3:
            clr(hn)

        def scan_c(j, carry):
            h = hc[pl.ds(j * 16, 16)]
            inc = plsc.cumsum(h)
            hc[pl.ds(j * 16, 16)] = inc - h + carry
            return carry + jnp.sum(h)
        lax.fori_loop(0, NBINS // 16, scan_c, jnp.int32(0))

        def scat_body(i, c):
            k = ki[pl.ds(i * 16, 16)]
            v = vi[pl.ds(i * 16, 16)]
            d = lax.shift_right_logical(k, shift) & 255
            cnt, _ = plsc.scan_count(d)
            cur = plsc.load_gather(hc, [d])
            pos = cur + cnt - 1
            plsc.store_scatter(ko, [pos], k)
            plsc.store_scatter(vo, [pos], v)
            plsc.addupdate_scatter(hc, [d], ones)
            if p < 3:
                dn = lax.shift_right_logical(k, shift + 8) & 255
                plsc.addupdate_scatter(hn, [dn], ones)
            return c
        lax.fori_loop(0, t2, scat_body, 0)

    pltpu.sync_copy(vb.at[pl.ds(0, NOUT)], out_hbm.at[wid])


def _make_sc_sort(interpret=False, **mesh_kw):
    return pl.kernel(
        _sc_sort_body,
        out_type=jax.ShapeDtypeStruct((NJOBS, NOUT), jnp.int32),
        mesh=plsc.VectorSubcoreMesh(
            core_axis_name="c", subcore_axis_name="s", num_cores=1,
            **mesh_kw),
        compiler_params=pltpu.CompilerParams(needs_layout_passes=False),
        interpret=interpret,
        scratch_types=[
            pltpu.VMEM((NQ * 4,), jnp.float32),
            pltpu.VMEM((NT * 4,), jnp.float32),
            pltpu.VMEM((NP,), jnp.int32),
            pltpu.VMEM((NP + 32,), jnp.int32),
            pltpu.VMEM((NP + 32,), jnp.int32),
            pltpu.VMEM((NP + 32,), jnp.int32),
            pltpu.VMEM((NP + 32,), jnp.int32),
            pltpu.VMEM((NBINS,), jnp.int32),
            pltpu.VMEM((NBINS,), jnp.int32),
        ],
    )


@functools.cache
def _get_sc_sort():
    return _make_sc_sort()


def kernel(logits, pred_boxes, target_boxes, class_labels):
    del logits, class_labels
    pb = pred_boxes.reshape(B, NQ * 4)
    tb = target_boxes.reshape(B, NT * 4)
    out = _get_sc_sort()(pb, tb)
    matched_pred = out[:B, :NM]
    matched_target = out[B:, :NM]
    return (matched_pred, matched_target)


# speedup vs baseline: 1.2489x; 1.2489x over previous
import functools

import numpy as np
import jax
import jax.numpy as jnp
from jax import lax
from jax.experimental import pallas as pl
from jax.experimental.pallas import tpu as pltpu
from jax.experimental.pallas import tpu_sc as plsc

B = 8
NQ = 5000
NT = 1000
NM = 1000
NP = 5008
NV = NP // 16
NOUT = 1024
NBINS = 256
NJOBS = 2 * B

_I32_MIN = np.int32(-(2**31))


def _area_key(x1, y1, x2, y2):
    a = (x2 - x1) * (y2 - y1)
    a = jnp.where(a == 0.0, 0.0, a)
    u = lax.bitcast_convert_type(a, jnp.int32)
    s = u >> 31
    m = u ^ (s | _I32_MIN)
    return ~m


def _sc_sort_body(pred_hbm, targ_hbm, out_hbm,
                  px1, py1, px2, py2, tx1, ty1, tx2, ty2,
                  kin, ka, kb, va, vb, hist, hist2):
    wid = lax.axis_index("s")
    lane = lax.iota(jnp.int32, 16)
    ones = jnp.ones(16, jnp.int32)
    zeros16 = jnp.zeros((16,), jnp.int32)
    neg16 = jnp.full((16,), -1, jnp.int32)

    def clr(h):
        def body(j, c):
            h[pl.ds(j * 16, 16)] = zeros16
            return c
        lax.fori_loop(0, NBINS // 16, body, 0)

    def build_keys(x1b, y1b, x2b, y2b, n_el):
        nfull = n_el // 16

        def body(i, c):
            key = _area_key(
                x1b[pl.ds(i * 16, 16)], y1b[pl.ds(i * 16, 16)],
                x2b[pl.ds(i * 16, 16)], y2b[pl.ds(i * 16, 16)])
            kin[pl.ds(i * 16, 16)] = key
            d = lax.shift_right_logical(key, 24)
            plsc.addupdate_scatter(hist, [d], ones)
            return c
        lax.fori_loop(0, nfull, body, 0)

        e = lane + nfull * 16
        valid = e < n_el
        idx = jnp.where(valid, e, 0)
        key = _area_key(
            plsc.load_gather(x1b, [idx]), plsc.load_gather(y1b, [idx]),
            plsc.load_gather(x2b, [idx]), plsc.load_gather(y2b, [idx]))
        key = jnp.where(valid, key, -1)
        kin[pl.ds(nfull * 16, 16)] = key
        d = lax.shift_right_logical(key, 24)
        plsc.addupdate_scatter(hist, [d], ones)

        def padbody(i, c):
            kin[pl.ds(i * 16, 16)] = neg16
            return c
        lax.fori_loop(nfull + 1, NV, padbody, 0)

    clr(hist)

    @pl.when(wid < B)
    def _():
        pltpu.sync_copy(pred_hbm.at[0, wid], px1)
        pltpu.sync_copy(pred_hbm.at[1, wid], py1)
        pltpu.sync_copy(pred_hbm.at[2, wid], px2)
        pltpu.sync_copy(pred_hbm.at[3, wid], py2)
        build_keys(px1, py1, px2, py2, NQ)

    @pl.when(wid >= B)
    def _():
        pltpu.sync_copy(targ_hbm.at[0, wid - B], tx1)
        pltpu.sync_copy(targ_hbm.at[1, wid - B], ty1)
        pltpu.sync_copy(targ_hbm.at[2, wid - B], tx2)
        pltpu.sync_copy(targ_hbm.at[3, wid - B], ty2)
        build_keys(tx1, ty1, tx2, ty2, NT)

    def scan_a(j, carry):
        tot, c = carry
        h = hist[pl.ds(j * 16, 16)]
        inc = plsc.cumsum(h) + tot
        c = c + jnp.sum(jnp.where(inc < NM, 1, 0))
        return (tot + jnp.sum(h), c)
    _, cut = lax.fori_loop(
        0, NBINS // 16, scan_a, (jnp.int32(0), jnp.int32(0)))

    clr(hist)

    def compact(i, off):
        k = kin[pl.ds(i * 16, 16)]
        d = lax.shift_right_logical(k, 24)
        m = d <= cut
        plsc.store_compressed(kb.at[pl.ds(off, 16)], k, mask=m)
        plsc.store_compressed(vb.at[pl.ds(off, 16)], lane + i * 16, mask=m)
        d0 = k & 255
        plsc.addupdate_scatter(hist, [d0], ones, mask=m)
        return off + jnp.sum(jnp.where(m, 1, 0))
    off = lax.fori_loop(0, NV, compact, jnp.int32(0))

    kb[pl.ds(off, 16)] = neg16
    vb[pl.ds(off, 16)] = lane + NP
    plsc.addupdate_scatter(hist, [jnp.full((16,), 255, jnp.int32)], ones)
    t2 = off // 16 + 1

    bufs = [
        (kb, vb, ka, va, hist, hist2),
        (ka, va, kb, vb, hist2, hist),
        (kb, vb, ka, va, hist, hist2),
        (ka, va, kb, vb, hist2, hist),
    ]
    for p, (ki, vi, ko, vo, hc, hn) in enumerate(bufs):
        shift = 8 * p
        if p < 3:
            clr(hn)

        def scan_c(j, carry):
            h = hc[pl.ds(j * 16, 16)]
            inc = plsc.cumsum(h)
            hc[pl.ds(j * 16, 16)] = inc - h + carry
            return carry + jnp.sum(h)
        lax.fori_loop(0, NBINS // 16, scan_c, jnp.int32(0))

        def scat_body(i, c):
            k = ki[pl.ds(i * 16, 16)]
            v = vi[pl.ds(i * 16, 16)]
            d = lax.shift_right_logical(k, shift) & 255
            cnt, _ = plsc.scan_count(d)
            cur = plsc.load_gather(hc, [d])
            pos = cur + cnt - 1
            plsc.store_scatter(ko, [pos], k)
            plsc.store_scatter(vo, [pos], v)
            plsc.addupdate_scatter(hc, [d], ones)
            if p < 3:
                dn = lax.shift_right_logical(k, shift + 8) & 255
                plsc.addupdate_scatter(hn, [dn], ones)
            return c
        lax.fori_loop(0, t2, scat_body, 0)

    pltpu.sync_copy(vb.at[pl.ds(0, NOUT)], out_hbm.at[wid])


def _make_sc_sort(interpret=False, **mesh_kw):
    return pl.kernel(
        _sc_sort_body,
        out_type=jax.ShapeDtypeStruct((NJOBS, NOUT), jnp.int32),
        mesh=plsc.VectorSubcoreMesh(
            core_axis_name="c", subcore_axis_name="s", num_cores=1,
            **mesh_kw),
        compiler_params=pltpu.CompilerParams(needs_layout_passes=False),
        interpret=interpret,
        scratch_types=[
            pltpu.VMEM((NQ,), jnp.float32),
            pltpu.VMEM((NQ,), jnp.float32),
            pltpu.VMEM((NQ,), jnp.float32),
            pltpu.VMEM((NQ,), jnp.float32),
            pltpu.VMEM((NT,), jnp.float32),
            pltpu.VMEM((NT,), jnp.float32),
            pltpu.VMEM((NT,), jnp.float32),
            pltpu.VMEM((NT,), jnp.float32),
            pltpu.VMEM((NP,), jnp.int32),
            pltpu.VMEM((NP + 32,), jnp.int32),
            pltpu.VMEM((NP + 32,), jnp.int32),
            pltpu.VMEM((NP + 32,), jnp.int32),
            pltpu.VMEM((NP + 32,), jnp.int32),
            pltpu.VMEM((NBINS,), jnp.int32),
            pltpu.VMEM((NBINS,), jnp.int32),
        ],
    )


@functools.cache
def _get_sc_sort():
    return _make_sc_sort()


def kernel(logits, pred_boxes, target_boxes, class_labels):
    del logits, class_labels
    pt = jnp.transpose(pred_boxes, (2, 0, 1))
    tt = jnp.transpose(target_boxes, (2, 0, 1))
    out = _get_sc_sort()(pt, tt)
    matched_pred = out[:B, :NM]
    matched_target = out[B:, :NM]
    return (matched_pred, matched_target)
